# Initial kernel scaffold; baseline (speedup 1.0000x reference)
import jax
import jax.numpy as jnp
from jax.experimental import pallas as pl


def kernel(x, edge_index, batch, enc_W, enc_b, W1, b1, W2, b2, eps, vn0, VW1, Vb1, VW2, Vb2, CW, Cb):
    raise NotImplementedError("write your pallas kernel here")


# speedup vs baseline: 4.6899x; 4.6899x over previous
import functools

import jax
import jax.numpy as jnp
from jax import lax
from jax.experimental import pallas as pl
from jax.experimental.pallas import tpu as pltpu
from jax.experimental.pallas import tpu_sc as plsc

F32 = jnp.float32
G = 128
ROWS = 2000


def _segsum_call(h, src, dst):
    N, D = h.shape
    E = src.shape[0]
    NC, NS = 2, 16
    per_tile = E // (NC * NS)
    CHUNK = 80
    n_chunks = per_tile // CHUNK
    NPAD = -(-N // (NS * 8)) * NS * 8
    rpt = NPAD // NS
    ZR = 128
    mesh = plsc.VectorSubcoreMesh(core_axis_name="c", subcore_axis_name="s")

    @functools.partial(
        pl.kernel,
        mesh=mesh,
        out_type=jax.ShapeDtypeStruct((NC, NPAD, D), F32),
        scratch_types=[
            pltpu.VMEM((CHUNK,), jnp.int32),
            pltpu.VMEM((CHUNK,), jnp.int32),
            pltpu.VMEM((CHUNK, D), F32),
            pltpu.VMEM((ZR, D), F32),
            pltpu.VMEM_SHARED((NPAD, D), F32),
            pltpu.SemaphoreType.DMA,
        ],
    )
    def segsum(h_hbm, src_hbm, dst_hbm, out_hbm, src_v, dst_v, rows_v, zbuf,
               acc_sh, sem):
        c = lax.axis_index("c")
        s = lax.axis_index("s")

        def zrow(i, carry):
            for j in range(D // 16):
                zbuf[i, pl.ds(j * 16, 16)] = jnp.zeros((16,), F32)
            return carry

        lax.fori_loop(0, ZR, zrow, 0)
        r0 = s * rpt
        for k in range(rpt // ZR):
            pltpu.sync_copy(zbuf, acc_sh.at[pl.ds(r0 + k * ZR, ZR)])
        rem = rpt % ZR
        if rem:
            pltpu.sync_copy(zbuf.at[pl.ds(0, rem)],
                            acc_sh.at[pl.ds(r0 + (rpt // ZR) * ZR, rem)])
        plsc.subcore_barrier()

        base0 = (c * NS + s) * per_tile

        def body(k, carry):
            base = base0 + k * CHUNK
            pltpu.sync_copy(src_hbm.at[pl.ds(base, CHUNK)], src_v)
            pltpu.sync_copy(dst_hbm.at[pl.ds(base, CHUNK)], dst_v)
            pltpu.async_copy(h_hbm.at[src_v], rows_v, sem).wait()
            pltpu.sync_copy(rows_v, acc_sh.at[dst_v], add=True)
            return carry

        lax.fori_loop(0, n_chunks, body, 0)
        plsc.subcore_barrier()
        pltpu.sync_copy(acc_sh.at[pl.ds(r0, rpt)],
                        out_hbm.at[c, pl.ds(r0, rpt)])

    return segsum(h, src, dst)


def _enc_body(x_ref, w_ref, b_ref, v_ref, o_ref):
    o_ref[...] = (jnp.dot(x_ref[...], w_ref[...], preferred_element_type=F32)
                  + b_ref[...] + v_ref[...])


def _enc_call(x, W, b, v):
    N, DI = x.shape
    EMB = W.shape[1]
    return pl.pallas_call(
        _enc_body,
        grid=(N // ROWS,),
        in_specs=[
            pl.BlockSpec((ROWS, DI), lambda i: (i, 0)),
            pl.BlockSpec((DI, EMB), lambda i: (0, 0)),
            pl.BlockSpec((1, EMB), lambda i: (0, 0)),
            pl.BlockSpec((1, EMB), lambda i: (0, 0)),
        ],
        out_specs=pl.BlockSpec((ROWS, EMB), lambda i: (i, 0)),
        out_shape=jax.ShapeDtypeStruct((N, EMB), F32),
    )(x, W, b, v)


def _onehot(bat_block):
    return (bat_block == lax.broadcasted_iota(jnp.int32, (ROWS, G), 1)
            ).astype(F32)


def _p1_body(hp, agg, bat, w1, b1, w2, b2, ep, zr, pooled):
    i = pl.program_id(0)
    a = agg[0] + agg[1]
    u = hp[...] * ep[...] + a
    t = jnp.maximum(jnp.dot(u, w1[...], preferred_element_type=F32)
                    + b1[...], 0.0)
    z = jnp.dot(t, w2[...], preferred_element_type=F32) + b2[...]
    z = jnp.maximum(z, 0.0)
    zr[...] = z
    oh = _onehot(bat[...])
    p = lax.dot_general(oh, z, (((0,), (0,)), ((), ())),
                        preferred_element_type=F32)

    @pl.when(i == 0)
    def _():
        pooled[...] = p

    @pl.when(i > 0)
    def _():
        pooled[...] += p


def _p1_call(hp, agg, bat, w1, b1, w2, b2, ep):
    N, EMB = hp.shape
    HID = w1.shape[1]
    return pl.pallas_call(
        _p1_body,
        grid=(N // ROWS,),
        in_specs=[
            pl.BlockSpec((ROWS, EMB), lambda i: (i, 0)),
            pl.BlockSpec((2, ROWS, EMB), lambda i: (0, i, 0)),
            pl.BlockSpec((ROWS, 1), lambda i: (i, 0)),
            pl.BlockSpec((EMB, HID), lambda i: (0, 0)),
            pl.BlockSpec((1, HID), lambda i: (0, 0)),
            pl.BlockSpec((HID, EMB), lambda i: (0, 0)),
            pl.BlockSpec((1, EMB), lambda i: (0, 0)),
            pl.BlockSpec((1, 1), lambda i: (0, 0)),
        ],
        out_specs=[
            pl.BlockSpec((ROWS, EMB), lambda i: (i, 0)),
            pl.BlockSpec((G, EMB), lambda i: (0, 0)),
        ],
        out_shape=[
            jax.ShapeDtypeStruct((N, EMB), F32),
            jax.ShapeDtypeStruct((G, EMB), F32),
        ],
    )(hp, agg, bat, w1, b1, w2, b2, ep)


def _p2_body(zr, pooled, vn, bat, vw1, vb1, vw2, vb2, vnn, hn, vns):
    i = pl.program_id(0)

    @pl.when(i == 0)
    def _():
        vt = pooled[...] + vn[...]
        t = jnp.maximum(jnp.dot(vt, vw1[...], preferred_element_type=F32)
                        + vb1[...], 0.0)
        v2 = jnp.maximum(jnp.dot(t, vw2[...], preferred_element_type=F32)
                         + vb2[...], 0.0)
        vns[...] = v2
        vnn[...] = v2

    oh = _onehot(bat[...])
    hn[...] = zr[...] + jnp.dot(oh, vns[...], preferred_element_type=F32)


def _p2_call(zr, pooled, vn, bat, vw1, vb1, vw2, vb2):
    N, EMB = zr.shape
    HID = vw1.shape[1]
    return pl.pallas_call(
        _p2_body,
        grid=(N // ROWS,),
        in_specs=[
            pl.BlockSpec((ROWS, EMB), lambda i: (i, 0)),
            pl.BlockSpec((G, EMB), lambda i: (0, 0)),
            pl.BlockSpec((G, EMB), lambda i: (0, 0)),
            pl.BlockSpec((ROWS, 1), lambda i: (i, 0)),
            pl.BlockSpec((EMB, HID), lambda i: (0, 0)),
            pl.BlockSpec((1, HID), lambda i: (0, 0)),
            pl.BlockSpec((HID, EMB), lambda i: (0, 0)),
            pl.BlockSpec((1, EMB), lambda i: (0, 0)),
        ],
        out_specs=[
            pl.BlockSpec((G, EMB), lambda i: (0, 0)),
            pl.BlockSpec((ROWS, EMB), lambda i: (i, 0)),
        ],
        out_shape=[
            jax.ShapeDtypeStruct((G, EMB), F32),
            jax.ShapeDtypeStruct((N, EMB), F32),
        ],
        scratch_shapes=[pltpu.VMEM((G, EMB), F32)],
    )(zr, pooled, vn, bat, vw1, vb1, vw2, vb2)


def _fin_body(hp, agg, bat, w1, b1, w2, b2, ep, cw, cb, pred, pooled_s, cnt_s):
    i = pl.program_id(0)
    a = agg[0] + agg[1]
    u = hp[...] * ep[...] + a
    t = jnp.maximum(jnp.dot(u, w1[...], preferred_element_type=F32)
                    + b1[...], 0.0)
    z = jnp.dot(t, w2[...], preferred_element_type=F32) + b2[...]
    oh = _onehot(bat[...])
    p = lax.dot_general(oh, z, (((0,), (0,)), ((), ())),
                        preferred_element_type=F32)
    cnt = lax.dot_general(oh, jnp.ones((ROWS, 8), F32),
                          (((0,), (0,)), ((), ())),
                          preferred_element_type=F32)

    @pl.when(i == 0)
    def _():
        pooled_s[...] = p
        cnt_s[...] = cnt

    @pl.when(i > 0)
    def _():
        pooled_s[...] += p
        cnt_s[...] += cnt

    @pl.when(i == pl.num_programs(0) - 1)
    def _():
        rep = pooled_s[...] / jnp.maximum(cnt_s[...][:, :1], 1.0)
        pred[...] = jnp.dot(rep, cw[...], preferred_element_type=F32) + cb[...]


def _fin_call(hp, agg, bat, w1, b1, w2, b2, ep, cw, cb):
    N, EMB = hp.shape
    HID = w1.shape[1]
    OUT = cw.shape[1]
    return pl.pallas_call(
        _fin_body,
        grid=(N // ROWS,),
        in_specs=[
            pl.BlockSpec((ROWS, EMB), lambda i: (i, 0)),
            pl.BlockSpec((2, ROWS, EMB), lambda i: (0, i, 0)),
            pl.BlockSpec((ROWS, 1), lambda i: (i, 0)),
            pl.BlockSpec((EMB, HID), lambda i: (0, 0)),
            pl.BlockSpec((1, HID), lambda i: (0, 0)),
            pl.BlockSpec((HID, EMB), lambda i: (0, 0)),
            pl.BlockSpec((1, EMB), lambda i: (0, 0)),
            pl.BlockSpec((1, 1), lambda i: (0, 0)),
            pl.BlockSpec((EMB, OUT), lambda i: (0, 0)),
            pl.BlockSpec((1, OUT), lambda i: (0, 0)),
        ],
        out_specs=pl.BlockSpec((G, OUT), lambda i: (0, 0)),
        out_shape=jax.ShapeDtypeStruct((G, OUT), F32),
        scratch_shapes=[pltpu.VMEM((G, EMB), F32), pltpu.VMEM((G, 8), F32)],
    )(hp, agg, bat, w1, b1, w2, b2, ep, cw, cb)


def kernel(x, edge_index, batch, enc_W, enc_b, W1, b1, W2, b2, eps, vn0,
           VW1, Vb1, VW2, Vb2, CW, Cb):
    N = x.shape[0]
    EMB = enc_W.shape[1]
    L = W1.shape[0]
    src = edge_index[0]
    dst = edge_index[1]
    bat = batch.reshape(N, 1)
    epsp = (1.0 + eps).reshape(L, 1, 1).astype(F32)

    h = _enc_call(x, enc_W, enc_b.reshape(1, EMB), vn0.reshape(1, EMB))
    vn = jnp.tile(vn0[None, :], (G, 1))
    for l in range(L - 1):
        agg = _segsum_call(h, src, dst)
        zr, pooled = _p1_call(h, agg, bat, W1[l], b1[l].reshape(1, -1),
                              W2[l], b2[l].reshape(1, -1), epsp[l])
        vn, h = _p2_call(zr, pooled, vn, bat, VW1[l], Vb1[l].reshape(1, -1),
                         VW2[l], Vb2[l].reshape(1, -1))
    agg = _segsum_call(h, src, dst)
    return _fin_call(h, agg, bat, W1[L - 1], b1[L - 1].reshape(1, -1),
                     W2[L - 1], b2[L - 1].reshape(1, -1), epsp[L - 1],
                     CW, Cb.reshape(1, -1))


# speedup vs baseline: 8.0436x; 1.7151x over previous
import functools

import jax
import jax.numpy as jnp
from jax import lax
from jax.experimental import pallas as pl
from jax.experimental.pallas import tpu as pltpu
from jax.experimental.pallas import tpu_sc as plsc

F32 = jnp.float32
G = 128
ROWS = 2000


def _segsum_call(h, src, dst):
    N, D = h.shape
    E = src.shape[0]
    NC, NS = 2, 16
    NW = NC * NS
    per_tile = E // NW
    CHUNK = 40
    NCH = per_tile // CHUNK
    NB = 5
    NPAD = -(-N // (NS * 8)) * NS * 8
    rpt = NPAD // NS
    mesh = plsc.VectorSubcoreMesh(core_axis_name="c", subcore_axis_name="s")

    src3 = src.reshape(NW * NCH, 1, CHUNK)
    dst3 = dst.reshape(NW * NCH, 1, CHUNK)

    @functools.partial(
        pl.kernel,
        mesh=mesh,
        out_type=jax.ShapeDtypeStruct((NC, NPAD, D), F32),
        scratch_types=[
            pltpu.VMEM((NB, 1, CHUNK), jnp.int32),
            pltpu.VMEM((NB, 1, CHUNK), jnp.int32),
            pltpu.VMEM((NB, CHUNK, D), F32),
            pltpu.VMEM_SHARED((NPAD, D), F32),
            pltpu.SemaphoreType.DMA((NB,)),
            pltpu.SemaphoreType.DMA((NB,)),
            pltpu.SemaphoreType.DMA((NB,)),
        ],
    )
    def segsum(h_hbm, src_hbm, dst_hbm, out_hbm, src_v, dst_v, rows_v,
               acc_sh, isem, gsem, ssem):
        c = lax.axis_index("c")
        s = lax.axis_index("s")
        w = c * NS + s

        def zrow(i, carry):
            for j in range(D // 16):
                rows_v[0, i, pl.ds(j * 16, 16)] = jnp.zeros((16,), F32)
            return carry

        lax.fori_loop(0, CHUNK, zrow, 0)
        r0 = s * rpt
        for k in range(rpt // CHUNK):
            pltpu.sync_copy(rows_v.at[0], acc_sh.at[pl.ds(r0 + k * CHUNK,
                                                          CHUNK)])
        plsc.subcore_barrier()

        def body(g, carry):
            k0 = w * NCH + g * NB
            icp = []
            for b in range(NB):
                icp.append(
                    (pltpu.async_copy(src_hbm.at[k0 + b], src_v.at[b],
                                      isem.at[b]),
                     pltpu.async_copy(dst_hbm.at[k0 + b], dst_v.at[b],
                                      isem.at[b])))
            gcp = []
            for b in range(NB):
                icp[b][0].wait()
                icp[b][1].wait()
                gcp.append(pltpu.async_copy(h_hbm.at[src_v.at[b, 0]],
                                            rows_v.at[b], gsem.at[b]))
            scp = []
            for b in range(NB):
                gcp[b].wait()
                scp.append(pltpu.async_copy(rows_v.at[b],
                                            acc_sh.at[dst_v.at[b, 0]],
                                            ssem.at[b], add=True))
            for cp in scp:
                cp.wait()
            return carry

        lax.fori_loop(0, NCH // NB, body, 0)
        plsc.subcore_barrier()
        pltpu.sync_copy(acc_sh.at[pl.ds(r0, rpt)],
                        out_hbm.at[c, pl.ds(r0, rpt)])

    return segsum(h, src3, dst3)


def _enc_body(x_ref, w_ref, b_ref, v_ref, o_ref):
    o_ref[...] = (jnp.dot(x_ref[...], w_ref[...], preferred_element_type=F32)
                  + b_ref[...] + v_ref[...])


def _enc_call(x, W, b, v):
    N, DI = x.shape
    EMB = W.shape[1]
    return pl.pallas_call(
        _enc_body,
        grid=(N // ROWS,),
        in_specs=[
            pl.BlockSpec((ROWS, DI), lambda i: (i, 0)),
            pl.BlockSpec((DI, EMB), lambda i: (0, 0)),
            pl.BlockSpec((1, EMB), lambda i: (0, 0)),
            pl.BlockSpec((1, EMB), lambda i: (0, 0)),
        ],
        out_specs=pl.BlockSpec((ROWS, EMB), lambda i: (i, 0)),
        out_shape=jax.ShapeDtypeStruct((N, EMB), F32),
    )(x, W, b, v)


def _onehot(bat_block):
    return (bat_block == lax.broadcasted_iota(jnp.int32, (ROWS, G), 1)
            ).astype(F32)


def _p1_body(hp, agg, bat, w1, b1, w2, b2, ep, zr, pooled):
    i = pl.program_id(0)
    a = agg[0] + agg[1]
    u = hp[...] * ep[...] + a
    t = jnp.maximum(jnp.dot(u, w1[...], preferred_element_type=F32)
                    + b1[...], 0.0)
    z = jnp.dot(t, w2[...], preferred_element_type=F32) + b2[...]
    z = jnp.maximum(z, 0.0)
    zr[...] = z
    oh = _onehot(bat[...])
    p = lax.dot_general(oh, z, (((0,), (0,)), ((), ())),
                        preferred_element_type=F32)

    @pl.when(i == 0)
    def _():
        pooled[...] = p

    @pl.when(i > 0)
    def _():
        pooled[...] += p


def _p1_call(hp, agg, bat, w1, b1, w2, b2, ep):
    N, EMB = hp.shape
    HID = w1.shape[1]
    return pl.pallas_call(
        _p1_body,
        grid=(N // ROWS,),
        in_specs=[
            pl.BlockSpec((ROWS, EMB), lambda i: (i, 0)),
            pl.BlockSpec((2, ROWS, EMB), lambda i: (0, i, 0)),
            pl.BlockSpec((ROWS, 1), lambda i: (i, 0)),
            pl.BlockSpec((EMB, HID), lambda i: (0, 0)),
            pl.BlockSpec((1, HID), lambda i: (0, 0)),
            pl.BlockSpec((HID, EMB), lambda i: (0, 0)),
            pl.BlockSpec((1, EMB), lambda i: (0, 0)),
            pl.BlockSpec((1, 1), lambda i: (0, 0)),
        ],
        out_specs=[
            pl.BlockSpec((ROWS, EMB), lambda i: (i, 0)),
            pl.BlockSpec((G, EMB), lambda i: (0, 0)),
        ],
        out_shape=[
            jax.ShapeDtypeStruct((N, EMB), F32),
            jax.ShapeDtypeStruct((G, EMB), F32),
        ],
    )(hp, agg, bat, w1, b1, w2, b2, ep)


def _p2_body(zr, pooled, vn, bat, vw1, vb1, vw2, vb2, vnn, hn, vns):
    i = pl.program_id(0)

    @pl.when(i == 0)
    def _():
        vt = pooled[...] + vn[...]
        t = jnp.maximum(jnp.dot(vt, vw1[...], preferred_element_type=F32)
                        + vb1[...], 0.0)
        v2 = jnp.maximum(jnp.dot(t, vw2[...], preferred_element_type=F32)
                         + vb2[...], 0.0)
        vns[...] = v2
        vnn[...] = v2

    oh = _onehot(bat[...])
    hn[...] = zr[...] + jnp.dot(oh, vns[...], preferred_element_type=F32)


def _p2_call(zr, pooled, vn, bat, vw1, vb1, vw2, vb2):
    N, EMB = zr.shape
    HID = vw1.shape[1]
    return pl.pallas_call(
        _p2_body,
        grid=(N // ROWS,),
        in_specs=[
            pl.BlockSpec((ROWS, EMB), lambda i: (i, 0)),
            pl.BlockSpec((G, EMB), lambda i: (0, 0)),
            pl.BlockSpec((G, EMB), lambda i: (0, 0)),
            pl.BlockSpec((ROWS, 1), lambda i: (i, 0)),
            pl.BlockSpec((EMB, HID), lambda i: (0, 0)),
            pl.BlockSpec((1, HID), lambda i: (0, 0)),
            pl.BlockSpec((HID, EMB), lambda i: (0, 0)),
            pl.BlockSpec((1, EMB), lambda i: (0, 0)),
        ],
        out_specs=[
            pl.BlockSpec((G, EMB), lambda i: (0, 0)),
            pl.BlockSpec((ROWS, EMB), lambda i: (i, 0)),
        ],
        out_shape=[
            jax.ShapeDtypeStruct((G, EMB), F32),
            jax.ShapeDtypeStruct((N, EMB), F32),
        ],
        scratch_shapes=[pltpu.VMEM((G, EMB), F32)],
    )(zr, pooled, vn, bat, vw1, vb1, vw2, vb2)


def _fin_body(hp, agg, bat, w1, b1, w2, b2, ep, cw, cb, pred, pooled_s, cnt_s):
    i = pl.program_id(0)
    a = agg[0] + agg[1]
    u = hp[...] * ep[...] + a
    t = jnp.maximum(jnp.dot(u, w1[...], preferred_element_type=F32)
                    + b1[...], 0.0)
    z = jnp.dot(t, w2[...], preferred_element_type=F32) + b2[...]
    oh = _onehot(bat[...])
    p = lax.dot_general(oh, z, (((0,), (0,)), ((), ())),
                        preferred_element_type=F32)
    cnt = lax.dot_general(oh, jnp.ones((ROWS, 8), F32),
                          (((0,), (0,)), ((), ())),
                          preferred_element_type=F32)

    @pl.when(i == 0)
    def _():
        pooled_s[...] = p
        cnt_s[...] = cnt

    @pl.when(i > 0)
    def _():
        pooled_s[...] += p
        cnt_s[...] += cnt

    @pl.when(i == pl.num_programs(0) - 1)
    def _():
        rep = pooled_s[...] / jnp.maximum(cnt_s[...][:, :1], 1.0)
        pred[...] = jnp.dot(rep, cw[...], preferred_element_type=F32) + cb[...]


def _fin_call(hp, agg, bat, w1, b1, w2, b2, ep, cw, cb):
    N, EMB = hp.shape
    HID = w1.shape[1]
    OUT = cw.shape[1]
    return pl.pallas_call(
        _fin_body,
        grid=(N // ROWS,),
        in_specs=[
            pl.BlockSpec((ROWS, EMB), lambda i: (i, 0)),
            pl.BlockSpec((2, ROWS, EMB), lambda i: (0, i, 0)),
            pl.BlockSpec((ROWS, 1), lambda i: (i, 0)),
            pl.BlockSpec((EMB, HID), lambda i: (0, 0)),
            pl.BlockSpec((1, HID), lambda i: (0, 0)),
            pl.BlockSpec((HID, EMB), lambda i: (0, 0)),
            pl.BlockSpec((1, EMB), lambda i: (0, 0)),
            pl.BlockSpec((1, 1), lambda i: (0, 0)),
            pl.BlockSpec((EMB, OUT), lambda i: (0, 0)),
            pl.BlockSpec((1, OUT), lambda i: (0, 0)),
        ],
        out_specs=pl.BlockSpec((G, OUT), lambda i: (0, 0)),
        out_shape=jax.ShapeDtypeStruct((G, OUT), F32),
        scratch_shapes=[pltpu.VMEM((G, EMB), F32), pltpu.VMEM((G, 8), F32)],
    )(hp, agg, bat, w1, b1, w2, b2, ep, cw, cb)


def kernel(x, edge_index, batch, enc_W, enc_b, W1, b1, W2, b2, eps, vn0,
           VW1, Vb1, VW2, Vb2, CW, Cb):
    N = x.shape[0]
    EMB = enc_W.shape[1]
    L = W1.shape[0]
    src = edge_index[0]
    dst = edge_index[1]
    bat = batch.reshape(N, 1)
    epsp = (1.0 + eps).reshape(L, 1, 1).astype(F32)

    h = _enc_call(x, enc_W, enc_b.reshape(1, EMB), vn0.reshape(1, EMB))
    vn = jnp.tile(vn0[None, :], (G, 1))
    for l in range(L - 1):
        agg = _segsum_call(h, src, dst)
        zr, pooled = _p1_call(h, agg, bat, W1[l], b1[l].reshape(1, -1),
                              W2[l], b2[l].reshape(1, -1), epsp[l])
        vn, h = _p2_call(zr, pooled, vn, bat, VW1[l], Vb1[l].reshape(1, -1),
                         VW2[l], Vb2[l].reshape(1, -1))
    agg = _segsum_call(h, src, dst)
    return _fin_call(h, agg, bat, W1[L - 1], b1[L - 1].reshape(1, -1),
                     W2[L - 1], b2[L - 1].reshape(1, -1), epsp[L - 1],
                     CW, Cb.reshape(1, -1))


# speedup vs baseline: 8.7265x; 1.0849x over previous
import functools

import jax
import jax.numpy as jnp
from jax import lax
from jax.experimental import pallas as pl
from jax.experimental.pallas import tpu as pltpu
from jax.experimental.pallas import tpu_sc as plsc

F32 = jnp.float32
G = 128
ROWS = 2000


def _segsum_call(h2, src, dst):
    _, N, DH = h2.shape
    E = src.shape[0]
    NC, NS = 2, 16
    per_tile = E // NS
    CHUNK = 80
    NCH = per_tile // CHUNK
    NB = 10
    NG = NCH // NB
    NPAD = -(-N // (NS * 8)) * NS * 8
    rpt = NPAD // NS
    mesh = plsc.VectorSubcoreMesh(core_axis_name="c", subcore_axis_name="s")

    srcR = src.reshape(1, NS * NG, NB, CHUNK)
    srcA = jnp.concatenate([srcR, srcR + N], axis=0).reshape(
        2 * NS * NG, NB, CHUNK)
    dstA = dst.reshape(NS * NG, NB, CHUNK)
    tab = h2.reshape(2 * N, DH)

    @functools.partial(
        pl.kernel,
        mesh=mesh,
        compiler_params=pltpu.CompilerParams(use_tc_tiling_on_sc=False),
        out_type=jax.ShapeDtypeStruct((NC, NPAD, DH), F32),
        scratch_types=[
            pltpu.VMEM((NB, CHUNK), jnp.int32),
            pltpu.VMEM((NB, CHUNK), jnp.int32),
            pltpu.VMEM((NB, CHUNK, DH), F32),
            pltpu.VMEM_SHARED((NPAD, DH), F32),
            pltpu.SemaphoreType.DMA,
            pltpu.SemaphoreType.DMA,
            pltpu.SemaphoreType.DMA((NB,)),
            pltpu.SemaphoreType.DMA((NB,)),
        ],
    )
    def segsum(tab_hbm, src_hbm, dst_hbm, out_hbm, src_v, dst_v, rows_v,
               acc_sh, isem0, isem1, gsem, ssem):
        c = lax.axis_index("c")
        s = lax.axis_index("s")

        def zrow(i, carry):
            for j in range(DH // 16):
                rows_v[0, i, pl.ds(j * 16, 16)] = jnp.zeros((16,), F32)
            return carry

        lax.fori_loop(0, CHUNK, zrow, 0)
        r0 = s * rpt
        for k in range(rpt // CHUNK):
            pltpu.sync_copy(rows_v.at[0], acc_sh.at[pl.ds(r0 + k * CHUNK,
                                                          CHUNK)])
        plsc.subcore_barrier()

        def body(g, carry):
            i1 = pltpu.async_copy(src_hbm.at[(c * NS + s) * NG + g], src_v,
                                  isem0)
            i2 = pltpu.async_copy(dst_hbm.at[s * NG + g], dst_v, isem1)
            i1.wait()
            i2.wait()
            gcp = []
            for b in range(NB):
                gcp.append(pltpu.async_copy(tab_hbm.at[src_v.at[b]],
                                            rows_v.at[b], gsem.at[b]))
            scp = []
            for b in range(NB):
                gcp[b].wait()
                scp.append(pltpu.async_copy(rows_v.at[b],
                                            acc_sh.at[dst_v.at[b]],
                                            ssem.at[b], add=True))
            for cp in scp:
                cp.wait()
            return carry

        lax.fori_loop(0, NG, body, 0)
        plsc.subcore_barrier()
        pltpu.sync_copy(acc_sh.at[pl.ds(r0, rpt)],
                        out_hbm.at[c, pl.ds(r0, rpt)])

    return segsum(tab, srcA, dstA)


def _split(ref, val):
    half = val.shape[1] // 2
    ref[0] = val[:, :half]
    ref[1] = val[:, half:]


def _enc_body(x_ref, w_ref, b_ref, v_ref, o_ref):
    _split(o_ref, jnp.dot(x_ref[...], w_ref[...], preferred_element_type=F32)
           + b_ref[...] + v_ref[...])


def _enc_call(x, W, b, v):
    N, DI = x.shape
    EMB = W.shape[1]
    return pl.pallas_call(
        _enc_body,
        grid=(N // ROWS,),
        in_specs=[
            pl.BlockSpec((ROWS, DI), lambda i: (i, 0)),
            pl.BlockSpec((DI, EMB), lambda i: (0, 0)),
            pl.BlockSpec((1, EMB), lambda i: (0, 0)),
            pl.BlockSpec((1, EMB), lambda i: (0, 0)),
        ],
        out_specs=pl.BlockSpec((2, ROWS, EMB // 2), lambda i: (0, i, 0)),
        out_shape=jax.ShapeDtypeStruct((2, N, EMB // 2), F32),
    )(x, W, b, v)


def _onehot(bat_block):
    return (bat_block == lax.broadcasted_iota(jnp.int32, (ROWS, G), 1)
            ).astype(F32)


def _p1_body(hp, agg, bat, w1, b1, w2, b2, ep, zr, pooled):
    i = pl.program_id(0)
    h = jnp.concatenate([hp[0], hp[1]], axis=1)
    a = jnp.concatenate([agg[0], agg[1]], axis=1)
    u = h * ep[...] + a
    t = jnp.maximum(jnp.dot(u, w1[...], preferred_element_type=F32)
                    + b1[...], 0.0)
    z = jnp.dot(t, w2[...], preferred_element_type=F32) + b2[...]
    z = jnp.maximum(z, 0.0)
    zr[...] = z
    oh = _onehot(bat[...])
    p = lax.dot_general(oh, z, (((0,), (0,)), ((), ())),
                        preferred_element_type=F32)

    @pl.when(i == 0)
    def _():
        pooled[...] = p

    @pl.when(i > 0)
    def _():
        pooled[...] += p


def _p1_call(hp, agg, bat, w1, b1, w2, b2, ep):
    _, N, HALF = hp.shape
    EMB = 2 * HALF
    HID = w1.shape[1]
    return pl.pallas_call(
        _p1_body,
        grid=(N // ROWS,),
        in_specs=[
            pl.BlockSpec((2, ROWS, HALF), lambda i: (0, i, 0)),
            pl.BlockSpec((2, ROWS, HALF), lambda i: (0, i, 0)),
            pl.BlockSpec((ROWS, 1), lambda i: (i, 0)),
            pl.BlockSpec((EMB, HID), lambda i: (0, 0)),
            pl.BlockSpec((1, HID), lambda i: (0, 0)),
            pl.BlockSpec((HID, EMB), lambda i: (0, 0)),
            pl.BlockSpec((1, EMB), lambda i: (0, 0)),
            pl.BlockSpec((1, 1), lambda i: (0, 0)),
        ],
        out_specs=[
            pl.BlockSpec((ROWS, EMB), lambda i: (i, 0)),
            pl.BlockSpec((G, EMB), lambda i: (0, 0)),
        ],
        out_shape=[
            jax.ShapeDtypeStruct((N, EMB), F32),
            jax.ShapeDtypeStruct((G, EMB), F32),
        ],
    )(hp, agg, bat, w1, b1, w2, b2, ep)


def _p2_body(zr, pooled, vn, bat, vw1, vb1, vw2, vb2, vnn, hn, vns):
    i = pl.program_id(0)

    @pl.when(i == 0)
    def _():
        vt = pooled[...] + vn[...]
        t = jnp.maximum(jnp.dot(vt, vw1[...], preferred_element_type=F32)
                        + vb1[...], 0.0)
        v2 = jnp.maximum(jnp.dot(t, vw2[...], preferred_element_type=F32)
                         + vb2[...], 0.0)
        vns[...] = v2
        vnn[...] = v2

    oh = _onehot(bat[...])
    _split(hn, zr[...] + jnp.dot(oh, vns[...], preferred_element_type=F32))


def _p2_call(zr, pooled, vn, bat, vw1, vb1, vw2, vb2):
    N, EMB = zr.shape
    HID = vw1.shape[1]
    return pl.pallas_call(
        _p2_body,
        grid=(N // ROWS,),
        in_specs=[
            pl.BlockSpec((ROWS, EMB), lambda i: (i, 0)),
            pl.BlockSpec((G, EMB), lambda i: (0, 0)),
            pl.BlockSpec((G, EMB), lambda i: (0, 0)),
            pl.BlockSpec((ROWS, 1), lambda i: (i, 0)),
            pl.BlockSpec((EMB, HID), lambda i: (0, 0)),
            pl.BlockSpec((1, HID), lambda i: (0, 0)),
            pl.BlockSpec((HID, EMB), lambda i: (0, 0)),
            pl.BlockSpec((1, EMB), lambda i: (0, 0)),
        ],
        out_specs=[
            pl.BlockSpec((G, EMB), lambda i: (0, 0)),
            pl.BlockSpec((2, ROWS, EMB // 2), lambda i: (0, i, 0)),
        ],
        out_shape=[
            jax.ShapeDtypeStruct((G, EMB), F32),
            jax.ShapeDtypeStruct((2, N, EMB // 2), F32),
        ],
        scratch_shapes=[pltpu.VMEM((G, EMB), F32)],
    )(zr, pooled, vn, bat, vw1, vb1, vw2, vb2)


def _fin_body(hp, agg, bat, w1, b1, w2, b2, ep, cw, cb, pred, pooled_s, cnt_s):
    i = pl.program_id(0)
    h = jnp.concatenate([hp[0], hp[1]], axis=1)
    a = jnp.concatenate([agg[0], agg[1]], axis=1)
    u = h * ep[...] + a
    t = jnp.maximum(jnp.dot(u, w1[...], preferred_element_type=F32)
                    + b1[...], 0.0)
    z = jnp.dot(t, w2[...], preferred_element_type=F32) + b2[...]
    oh = _onehot(bat[...])
    p = lax.dot_general(oh, z, (((0,), (0,)), ((), ())),
                        preferred_element_type=F32)
    cnt = lax.dot_general(oh, jnp.ones((ROWS, 8), F32),
                          (((0,), (0,)), ((), ())),
                          preferred_element_type=F32)

    @pl.when(i == 0)
    def _():
        pooled_s[...] = p
        cnt_s[...] = cnt

    @pl.when(i > 0)
    def _():
        pooled_s[...] += p
        cnt_s[...] += cnt

    @pl.when(i == pl.num_programs(0) - 1)
    def _():
        rep = pooled_s[...] / jnp.maximum(cnt_s[...][:, :1], 1.0)
        pred[...] = jnp.dot(rep, cw[...], preferred_element_type=F32) + cb[...]


def _fin_call(hp, agg, bat, w1, b1, w2, b2, ep, cw, cb):
    _, N, HALF = hp.shape
    EMB = 2 * HALF
    HID = w1.shape[1]
    OUT = cw.shape[1]
    return pl.pallas_call(
        _fin_body,
        grid=(N // ROWS,),
        in_specs=[
            pl.BlockSpec((2, ROWS, HALF), lambda i: (0, i, 0)),
            pl.BlockSpec((2, ROWS, HALF), lambda i: (0, i, 0)),
            pl.BlockSpec((ROWS, 1), lambda i: (i, 0)),
            pl.BlockSpec((EMB, HID), lambda i: (0, 0)),
            pl.BlockSpec((1, HID), lambda i: (0, 0)),
            pl.BlockSpec((HID, EMB), lambda i: (0, 0)),
            pl.BlockSpec((1, EMB), lambda i: (0, 0)),
            pl.BlockSpec((1, 1), lambda i: (0, 0)),
            pl.BlockSpec((EMB, OUT), lambda i: (0, 0)),
            pl.BlockSpec((1, OUT), lambda i: (0, 0)),
        ],
        out_specs=pl.BlockSpec((G, OUT), lambda i: (0, 0)),
        out_shape=jax.ShapeDtypeStruct((G, OUT), F32),
        scratch_shapes=[pltpu.VMEM((G, EMB), F32), pltpu.VMEM((G, 8), F32)],
    )(hp, agg, bat, w1, b1, w2, b2, ep, cw, cb)


def kernel(x, edge_index, batch, enc_W, enc_b, W1, b1, W2, b2, eps, vn0,
           VW1, Vb1, VW2, Vb2, CW, Cb):
    N = x.shape[0]
    EMB = enc_W.shape[1]
    L = W1.shape[0]
    src = edge_index[0]
    dst = edge_index[1]
    bat = batch.reshape(N, 1)
    epsp = (1.0 + eps).reshape(L, 1, 1).astype(F32)

    h = _enc_call(x, enc_W, enc_b.reshape(1, EMB), vn0.reshape(1, EMB))
    vn = jnp.tile(vn0[None, :], (G, 1))
    for l in range(L - 1):
        agg = _segsum_call(h, src, dst)
        zr, pooled = _p1_call(h, agg, bat, W1[l], b1[l].reshape(1, -1),
                              W2[l], b2[l].reshape(1, -1), epsp[l])
        vn, h = _p2_call(zr, pooled, vn, bat, VW1[l], Vb1[l].reshape(1, -1),
                         VW2[l], Vb2[l].reshape(1, -1))
    agg = _segsum_call(h, src, dst)
    return _fin_call(h, agg, bat, W1[L - 1], b1[L - 1].reshape(1, -1),
                     W2[L - 1], b2[L - 1].reshape(1, -1), epsp[L - 1],
                     CW, Cb.reshape(1, -1))


# speedup vs baseline: 8.8853x; 1.0182x over previous
import functools

import jax
import jax.numpy as jnp
from jax import lax
from jax.experimental import pallas as pl
from jax.experimental.pallas import tpu as pltpu
from jax.experimental.pallas import tpu_sc as plsc

F32 = jnp.float32
G = 128
ROWS = 2000


def _segsum_call(h2, src, dst):
    _, N, DH = h2.shape
    E = src.shape[0]
    NC, NS = 2, 16
    per_tile = E // NS
    CHUNK = 80
    NCH = per_tile // CHUNK
    NB = 10
    NG = NCH // NB
    NPAD = -(-N // (NS * 8)) * NS * 8
    rpt = NPAD // NS
    mesh = plsc.VectorSubcoreMesh(core_axis_name="c", subcore_axis_name="s")

    srcR = src.reshape(1, NS * NG, NB, CHUNK)
    srcA = jnp.concatenate([srcR, srcR + N], axis=0).reshape(
        2 * NS * NG, NB, CHUNK)
    dstA = dst.reshape(NS * NG, NB, CHUNK)
    tab = h2.reshape(2 * N, DH)

    @functools.partial(
        pl.kernel,
        mesh=mesh,
        compiler_params=pltpu.CompilerParams(use_tc_tiling_on_sc=False),
        out_type=jax.ShapeDtypeStruct((NC, NPAD, DH), F32),
        scratch_types=[
            pltpu.VMEM((NB, CHUNK), jnp.int32),
            pltpu.VMEM((NB, CHUNK), jnp.int32),
            pltpu.VMEM((NB, CHUNK, DH), F32),
            pltpu.VMEM_SHARED((NPAD, DH), F32),
            pltpu.SemaphoreType.DMA,
            pltpu.SemaphoreType.DMA,
            pltpu.SemaphoreType.DMA((NB,)),
            pltpu.SemaphoreType.DMA((NB,)),
        ],
    )
    def segsum(tab_hbm, src_hbm, dst_hbm, out_hbm, src_v, dst_v, rows_v,
               acc_sh, isem0, isem1, gsem, ssem):
        c = lax.axis_index("c")
        s = lax.axis_index("s")

        def zrow(i, carry):
            for j in range(DH // 16):
                rows_v[0, i, pl.ds(j * 16, 16)] = jnp.zeros((16,), F32)
            return carry

        lax.fori_loop(0, CHUNK, zrow, 0)
        r0 = s * rpt
        for k in range(rpt // CHUNK):
            pltpu.sync_copy(rows_v.at[0], acc_sh.at[pl.ds(r0 + k * CHUNK,
                                                          CHUNK)])
        plsc.subcore_barrier()

        def body(g, carry):
            i1 = pltpu.async_copy(src_hbm.at[(c * NS + s) * NG + g], src_v,
                                  isem0)
            i2 = pltpu.async_copy(dst_hbm.at[s * NG + g], dst_v, isem1)
            i1.wait()
            i2.wait()
            gcp = []
            for b in range(NB):
                gcp.append(pltpu.async_copy(tab_hbm.at[src_v.at[b]],
                                            rows_v.at[b], gsem.at[b]))
            scp = []
            for b in range(NB):
                gcp[b].wait()
                scp.append(pltpu.async_copy(rows_v.at[b],
                                            acc_sh.at[dst_v.at[b]],
                                            ssem.at[b], add=True))
            for cp in scp:
                cp.wait()
            return carry

        lax.fori_loop(0, NG, body, 0)
        plsc.subcore_barrier()
        pltpu.sync_copy(acc_sh.at[pl.ds(r0, rpt)],
                        out_hbm.at[c, pl.ds(r0, rpt)])

    return segsum(tab, srcA, dstA)


def _split(ref, val):
    half = val.shape[1] // 2
    ref[0] = val[:, :half]
    ref[1] = val[:, half:]


def _enc_body(x_ref, w_ref, b_ref, v_ref, o_ref):
    _split(o_ref, jnp.dot(x_ref[...], w_ref[...], preferred_element_type=F32)
           + b_ref[...] + v_ref[...])


def _enc_call(x, W, b, v):
    N, DI = x.shape
    EMB = W.shape[1]
    return pl.pallas_call(
        _enc_body,
        grid=(N // ROWS,),
        in_specs=[
            pl.BlockSpec((ROWS, DI), lambda i: (i, 0)),
            pl.BlockSpec((DI, EMB), lambda i: (0, 0)),
            pl.BlockSpec((1, EMB), lambda i: (0, 0)),
            pl.BlockSpec((1, EMB), lambda i: (0, 0)),
        ],
        out_specs=pl.BlockSpec((2, ROWS, EMB // 2), lambda i: (0, i, 0)),
        out_shape=jax.ShapeDtypeStruct((2, N, EMB // 2), F32),
    )(x, W, b, v)


def _onehot(bat_block):
    return (bat_block == lax.broadcasted_iota(jnp.int32, (ROWS, G), 1)
            ).astype(F32)


def _layer_body(hp, agg, bat, vn, w1, b1, w2, b2, ep, vw1, vb1, vw2, vb2,
                vnn, hn, zbuf, pooled, vns):
    p = pl.program_id(0)
    i = pl.program_id(1)
    oh = _onehot(bat[...])

    @pl.when(p == 0)
    def _():
        h = jnp.concatenate([hp[0], hp[1]], axis=1)
        a = jnp.concatenate([agg[0], agg[1]], axis=1)
        u = h * ep[...] + a
        t = jnp.maximum(jnp.dot(u, w1[...], preferred_element_type=F32)
                        + b1[...], 0.0)
        z = jnp.dot(t, w2[...], preferred_element_type=F32) + b2[...]
        z = jnp.maximum(z, 0.0)
        zbuf[pl.ds(i * ROWS, ROWS), :] = z
        pp = lax.dot_general(oh, z, (((0,), (0,)), ((), ())),
                             preferred_element_type=F32)

        @pl.when(i == 0)
        def _():
            pooled[...] = pp

        @pl.when(i > 0)
        def _():
            pooled[...] += pp

    @pl.when((p == 1) & (i == 0))
    def _():
        vt = pooled[...] + vn[...]
        t = jnp.maximum(jnp.dot(vt, vw1[...], preferred_element_type=F32)
                        + vb1[...], 0.0)
        v2 = jnp.maximum(jnp.dot(t, vw2[...], preferred_element_type=F32)
                         + vb2[...], 0.0)
        vns[...] = v2
        vnn[...] = v2

    @pl.when(p == 1)
    def _():
        z = zbuf[pl.ds(i * ROWS, ROWS), :]
        _split(hn, z + jnp.dot(oh, vns[...], preferred_element_type=F32))


def _layer_call(hp, agg, bat, vn, w1, b1, w2, b2, ep, vw1, vb1, vw2, vb2):
    _, N, HALF = hp.shape
    EMB = 2 * HALF
    HID = w1.shape[1]
    return pl.pallas_call(
        _layer_body,
        grid=(2, N // ROWS),
        in_specs=[
            pl.BlockSpec((2, ROWS, HALF),
                         lambda p, i: (0, jnp.where(p == 0, i, 0), 0)),
            pl.BlockSpec((2, ROWS, HALF),
                         lambda p, i: (0, jnp.where(p == 0, i, 0), 0)),
            pl.BlockSpec((ROWS, 1), lambda p, i: (i, 0)),
            pl.BlockSpec((G, EMB), lambda p, i: (0, 0)),
            pl.BlockSpec((EMB, HID), lambda p, i: (0, 0)),
            pl.BlockSpec((1, HID), lambda p, i: (0, 0)),
            pl.BlockSpec((HID, EMB), lambda p, i: (0, 0)),
            pl.BlockSpec((1, EMB), lambda p, i: (0, 0)),
            pl.BlockSpec((1, 1), lambda p, i: (0, 0)),
            pl.BlockSpec((EMB, HID), lambda p, i: (0, 0)),
            pl.BlockSpec((1, HID), lambda p, i: (0, 0)),
            pl.BlockSpec((HID, EMB), lambda p, i: (0, 0)),
            pl.BlockSpec((1, EMB), lambda p, i: (0, 0)),
        ],
        out_specs=[
            pl.BlockSpec((G, EMB), lambda p, i: (0, 0)),
            pl.BlockSpec((2, ROWS, HALF),
                         lambda p, i: (0, jnp.where(p == 1, i, 0), 0)),
        ],
        out_shape=[
            jax.ShapeDtypeStruct((G, EMB), F32),
            jax.ShapeDtypeStruct((2, N, HALF), F32),
        ],
        scratch_shapes=[pltpu.VMEM((N, EMB), F32), pltpu.VMEM((G, EMB), F32),
                        pltpu.VMEM((G, EMB), F32)],
    )(hp, agg, bat, vn, w1, b1, w2, b2, ep, vw1, vb1, vw2, vb2)


def _fin_body(hp, agg, bat, w1, b1, w2, b2, ep, cw, cb, pred, pooled_s, cnt_s):
    i = pl.program_id(0)
    h = jnp.concatenate([hp[0], hp[1]], axis=1)
    a = jnp.concatenate([agg[0], agg[1]], axis=1)
    u = h * ep[...] + a
    t = jnp.maximum(jnp.dot(u, w1[...], preferred_element_type=F32)
                    + b1[...], 0.0)
    z = jnp.dot(t, w2[...], preferred_element_type=F32) + b2[...]
    oh = _onehot(bat[...])
    p = lax.dot_general(oh, z, (((0,), (0,)), ((), ())),
                        preferred_element_type=F32)
    cnt = lax.dot_general(oh, jnp.ones((ROWS, 8), F32),
                          (((0,), (0,)), ((), ())),
                          preferred_element_type=F32)

    @pl.when(i == 0)
    def _():
        pooled_s[...] = p
        cnt_s[...] = cnt

    @pl.when(i > 0)
    def _():
        pooled_s[...] += p
        cnt_s[...] += cnt

    @pl.when(i == pl.num_programs(0) - 1)
    def _():
        rep = pooled_s[...] / jnp.maximum(cnt_s[...][:, :1], 1.0)
        pred[...] = jnp.dot(rep, cw[...], preferred_element_type=F32) + cb[...]


def _fin_call(hp, agg, bat, w1, b1, w2, b2, ep, cw, cb):
    _, N, HALF = hp.shape
    EMB = 2 * HALF
    HID = w1.shape[1]
    OUT = cw.shape[1]
    return pl.pallas_call(
        _fin_body,
        grid=(N // ROWS,),
        in_specs=[
            pl.BlockSpec((2, ROWS, HALF), lambda i: (0, i, 0)),
            pl.BlockSpec((2, ROWS, HALF), lambda i: (0, i, 0)),
            pl.BlockSpec((ROWS, 1), lambda i: (i, 0)),
            pl.BlockSpec((EMB, HID), lambda i: (0, 0)),
            pl.BlockSpec((1, HID), lambda i: (0, 0)),
            pl.BlockSpec((HID, EMB), lambda i: (0, 0)),
            pl.BlockSpec((1, EMB), lambda i: (0, 0)),
            pl.BlockSpec((1, 1), lambda i: (0, 0)),
            pl.BlockSpec((EMB, OUT), lambda i: (0, 0)),
            pl.BlockSpec((1, OUT), lambda i: (0, 0)),
        ],
        out_specs=pl.BlockSpec((G, OUT), lambda i: (0, 0)),
        out_shape=jax.ShapeDtypeStruct((G, OUT), F32),
        scratch_shapes=[pltpu.VMEM((G, EMB), F32), pltpu.VMEM((G, 8), F32)],
    )(hp, agg, bat, w1, b1, w2, b2, ep, cw, cb)


def kernel(x, edge_index, batch, enc_W, enc_b, W1, b1, W2, b2, eps, vn0,
           VW1, Vb1, VW2, Vb2, CW, Cb):
    N = x.shape[0]
    EMB = enc_W.shape[1]
    L = W1.shape[0]
    src = edge_index[0]
    dst = edge_index[1]
    bat = batch.reshape(N, 1)
    epsp = (1.0 + eps).reshape(L, 1, 1).astype(F32)

    h = _enc_call(x, enc_W, enc_b.reshape(1, EMB), vn0.reshape(1, EMB))
    vn = jnp.tile(vn0[None, :], (G, 1))
    for l in range(L - 1):
        agg = _segsum_call(h, src, dst)
        vn, h = _layer_call(h, agg, bat, vn, W1[l], b1[l].reshape(1, -1),
                            W2[l], b2[l].reshape(1, -1), epsp[l],
                            VW1[l], Vb1[l].reshape(1, -1),
                            VW2[l], Vb2[l].reshape(1, -1))
    agg = _segsum_call(h, src, dst)
    return _fin_call(h, agg, bat, W1[L - 1], b1[L - 1].reshape(1, -1),
                     W2[L - 1], b2[L - 1].reshape(1, -1), epsp[L - 1],
                     CW, Cb.reshape(1, -1))
